# baseline (speedup 1.0000x reference)
import functools

import jax
import jax.numpy as jnp
from jax import lax
from jax.experimental import pallas as pl
from jax.experimental.pallas import tpu as pltpu
from jax.experimental.pallas import tpu_sc as plsc

NC = 2
NS = 16
NW = NC * NS
K = 80
PN = 10240
RPW = PN // NS

_mesh = plsc.VectorSubcoreMesh(core_axis_name="c", subcore_axis_name="s")


def _deg_sc(dst3, n_nodes):
    nw, j_batches, k = dst3.shape

    @functools.partial(
        pl.kernel,
        mesh=_mesh,
        out_type=jax.ShapeDtypeStruct((NC, PN), jnp.float32),
        scratch_types=[
            pltpu.VMEM((j_batches, k), jnp.int32),
            pltpu.VMEM((k,), jnp.float32),
            pltpu.VMEM((RPW,), jnp.float32),
            pltpu.VMEM_SHARED((PN,), jnp.float32),
        ],
    )
    def deg_k(dst_hbm, outd, dst_slab, ones_v, zero_v, accd):
        cid = lax.axis_index("c")
        sid = lax.axis_index("s")
        wid = sid * NC + cid
        for t in range(k // 16):
            ones_v[pl.ds(t * 16, 16)] = jnp.ones((16,), jnp.float32)
        for t in range(RPW // 16):
            zero_v[pl.ds(t * 16, 16)] = jnp.zeros((16,), jnp.float32)
        base = pl.multiple_of(sid * RPW, 8)
        pltpu.sync_copy(zero_v, accd.at[pl.ds(base, RPW)])
        plsc.subcore_barrier()
        pltpu.sync_copy(dst_hbm.at[wid], dst_slab)

        def body(j, carry):
            pltpu.sync_copy(ones_v, accd.at[dst_slab.at[j]], add=True)
            return carry

        lax.fori_loop(0, j_batches, body, 0)
        plsc.subcore_barrier()
        pltpu.sync_copy(accd.at[pl.ds(base, RPW)], outd.at[cid, pl.ds(base, RPW)])

    return deg_k(dst3)


def _msg_sc(hp, src3, dst3):
    nwc, chunk, k = src3.shape
    n_chunks = nwc // NW
    d_model = hp.shape[1]
    assert chunk % 4 == 0 and chunk >= 8

    @functools.partial(
        pl.kernel,
        mesh=_mesh,
        out_type=jax.ShapeDtypeStruct((NC, PN, d_model), jnp.float32),
        scratch_types=[
            pltpu.VMEM((chunk, k), jnp.int32),
            pltpu.VMEM((chunk, k), jnp.int32),
            pltpu.VMEM((k, d_model), jnp.float32),
            pltpu.VMEM((k, d_model), jnp.float32),
            pltpu.VMEM((k, d_model), jnp.float32),
            pltpu.VMEM((k, d_model), jnp.float32),
            pltpu.VMEM((16, d_model), jnp.float32),
            pltpu.VMEM_SHARED((PN, d_model), jnp.float32),
        ] + [pltpu.SemaphoreType.DMA] * 8,
    )
    def msg_k(hp_hbm, src_hbm, dst_hbm, outp, src_slab, dst_slab, b0, b1, b2,
              b3, zrow, acc, sg0, sg1, sg2, sg3, ss0, ss1, ss2, ss3):
        cid = lax.axis_index("c")
        sid = lax.axis_index("s")
        wid = sid * NC + cid
        bufs = (b0, b1, b2, b3)
        sg = (sg0, sg1, sg2, sg3)
        ss = (ss0, ss1, ss2, ss3)
        for r in range(16):
            for t in range(d_model // 16):
                zrow[r, pl.ds(t * 16, 16)] = jnp.zeros((16,), jnp.float32)
        base = pl.multiple_of(sid * RPW, 8)

        def zb(t, carry):
            pltpu.sync_copy(zrow, acc.at[pl.ds(base + t * 16, 16)])
            return carry

        lax.fori_loop(0, RPW // 16, zb, 0)
        plsc.subcore_barrier()

        def fire_g(j, u):
            pltpu.async_copy(hp_hbm.at[src_slab.at[j]], bufs[u], sg[u])

        def drain_g(j, u):
            pltpu.make_async_copy(hp_hbm.at[src_slab.at[j]], bufs[u], sg[u]).wait()

        def fire_s(j, u):
            pltpu.async_copy(bufs[u], acc.at[dst_slab.at[j]], ss[u], add=True)

        def drain_s(j, u):
            pltpu.make_async_copy(bufs[u], acc.at[dst_slab.at[j]], ss[u]).wait()

        def run_chunk(c, carry):
            pltpu.sync_copy(src_hbm.at[wid * n_chunks + c], src_slab)
            pltpu.sync_copy(dst_hbm.at[wid * n_chunks + c], dst_slab)
            fire_g(0, 0)
            fire_g(1, 1)
            drain_g(0, 0)
            fire_s(0, 0)
            fire_g(2, 2)
            drain_g(1, 1)
            fire_s(1, 1)
            fire_g(3, 3)

            def body(t, inner):
                for u in range(4):
                    j = 4 * t + 2 + u
                    bu = (2 + u) % 4
                    drain_g(j, bu)
                    fire_s(j, bu)
                    drain_s(j - 2, u % 4)
                    fire_g(j + 2, u % 4)
                return inner

            lax.fori_loop(0, (chunk - 4) // 4, body, 0)
            drain_g(chunk - 2, 2)
            fire_s(chunk - 2, 2)
            drain_s(chunk - 4, 0)
            drain_g(chunk - 1, 3)
            fire_s(chunk - 1, 3)
            drain_s(chunk - 3, 1)
            drain_s(chunk - 2, 2)
            drain_s(chunk - 1, 3)
            return carry

        lax.fori_loop(0, n_chunks, run_chunk, 0)
        plsc.subcore_barrier()
        pltpu.sync_copy(acc.at[pl.ds(base, RPW)], outp.at[cid, pl.ds(base, RPW)])

    return msg_k(hp, src3, dst3)


def _lin_tc(x, w):
    n, d_in = x.shape
    d_out = w.shape[0]

    def body(x_ref, w_ref, o_ref):
        o_ref[...] = lax.dot_general(
            x_ref[...], w_ref[...], (((1,), (1,)), ((), ())),
            preferred_element_type=jnp.float32)

    return pl.pallas_call(
        body, out_shape=jax.ShapeDtypeStruct((n, d_out), jnp.float32)
    )(x, w)


def _scale_tc(h, dp):
    n, d_out = h.shape

    def body(h_ref, dp_ref, o_ref):
        deg = jnp.sum(dp_ref[...], axis=1, keepdims=True) + 1.0
        o_ref[...] = h_ref[...] * lax.rsqrt(deg)

    return pl.pallas_call(
        body, out_shape=jax.ShapeDtypeStruct((n, d_out), jnp.float32)
    )(h, dp)


def _final_tc(p0, p1, hp, dp, bias2):
    n, d_model = hp.shape

    def body(p0_ref, p1_ref, hp_ref, dp_ref, b_ref, o_ref):
        deg = jnp.sum(dp_ref[...], axis=1, keepdims=True) + 1.0
        dis = lax.rsqrt(deg)
        o_ref[...] = dis * (p0_ref[...] + p1_ref[...] + hp_ref[...]) + b_ref[...]

    return pl.pallas_call(
        body, out_shape=jax.ShapeDtypeStruct((n, d_model), jnp.float32)
    )(p0, p1, hp, dp, bias2)


def kernel(x, edge_index, W, bias):
    n, d_in = x.shape
    e = edge_index.shape[1]
    assert e % (NW * K) == 0, (e, NW, K)
    j_batches = e // (NW * K)
    src3 = edge_index[0].reshape(NW, j_batches, K)
    dst3 = edge_index[1].reshape(NW, j_batches, K)

    h = _lin_tc(x, W)
    degp = _deg_sc(dst3, n)
    dp = degp[:, :n].T
    hp = _scale_tc(h, dp)
    km, chunk = 50, 40
    assert e % (NW * km * chunk) == 0
    src4 = edge_index[0].reshape(NW * (e // (NW * km * chunk)), chunk, km)
    dst4 = edge_index[1].reshape(NW * (e // (NW * km * chunk)), chunk, km)
    parts = _msg_sc(hp, src4, dst4)
    return _final_tc(parts[0, :n], parts[1, :n], hp, dp,
                     bias.reshape(1, -1).astype(jnp.float32))


# speedup vs baseline: 1.1323x; 1.1323x over previous
import functools

import jax
import jax.numpy as jnp
from jax import lax
from jax.experimental import pallas as pl
from jax.experimental.pallas import tpu as pltpu
from jax.experimental.pallas import tpu_sc as plsc

NC = 2
NS = 16
NW = NC * NS
K = 80
PN = 10240
RPW = PN // NS

_mesh = plsc.VectorSubcoreMesh(core_axis_name="c", subcore_axis_name="s")


def _deg_sc(dst3, n_nodes):
    nw, j_batches, k = dst3.shape

    @functools.partial(
        pl.kernel,
        mesh=_mesh,
        out_type=jax.ShapeDtypeStruct((NC, PN), jnp.float32),
        scratch_types=[
            pltpu.VMEM((j_batches, k), jnp.int32),
            pltpu.VMEM((k,), jnp.float32),
            pltpu.VMEM((RPW,), jnp.float32),
            pltpu.VMEM_SHARED((PN,), jnp.float32),
            pltpu.SemaphoreType.DMA,
        ],
    )
    def deg_k(dst_hbm, outd, dst_slab, ones_v, zero_v, accd, sem):
        cid = lax.axis_index("c")
        sid = lax.axis_index("s")
        wid = sid * NC + cid
        for t in range(k // 16):
            ones_v[pl.ds(t * 16, 16)] = jnp.ones((16,), jnp.float32)
        for t in range(RPW // 16):
            zero_v[pl.ds(t * 16, 16)] = jnp.zeros((16,), jnp.float32)
        base = pl.multiple_of(sid * RPW, 8)
        pltpu.sync_copy(zero_v, accd.at[pl.ds(base, RPW)])
        plsc.subcore_barrier()
        pltpu.sync_copy(dst_hbm.at[wid], dst_slab)

        lag = 16

        def body(j, carry):
            pltpu.async_copy(ones_v, accd.at[dst_slab.at[j]], sem, add=True)

            @pl.when(j >= lag)
            def _():
                pltpu.make_async_copy(
                    ones_v, accd.at[dst_slab.at[j - lag]], sem).wait()

            return carry

        lax.fori_loop(0, j_batches, body, 0)

        def drain(j, carry):
            pltpu.make_async_copy(
                ones_v, accd.at[dst_slab.at[j]], sem).wait()
            return carry

        lax.fori_loop(j_batches - lag, j_batches, drain, 0)
        plsc.subcore_barrier()
        pltpu.sync_copy(accd.at[pl.ds(base, RPW)], outd.at[cid, pl.ds(base, RPW)])

    return deg_k(dst3)


def _msg_sc(hp, src3, dst3):
    nwc, chunk, k = src3.shape
    n_chunks = nwc // NW
    d_model = hp.shape[1]
    assert chunk % 2 == 1

    @functools.partial(
        pl.kernel,
        mesh=_mesh,
        out_type=jax.ShapeDtypeStruct((NC, PN, d_model), jnp.float32),
        scratch_types=[
            pltpu.VMEM((chunk, k), jnp.int32),
            pltpu.VMEM((chunk, k), jnp.int32),
            pltpu.VMEM((k, d_model), jnp.float32),
            pltpu.VMEM((k, d_model), jnp.float32),
            pltpu.VMEM((16, d_model), jnp.float32),
            pltpu.VMEM_SHARED((PN, d_model), jnp.float32),
            pltpu.SemaphoreType.DMA,
            pltpu.SemaphoreType.DMA,
        ],
    )
    def msg_k(hp_hbm, src_hbm, dst_hbm, outp, src_slab, dst_slab, rows0,
              rows1, zrow, acc, sem0, sem1):
        cid = lax.axis_index("c")
        sid = lax.axis_index("s")
        wid = sid * NC + cid
        for r in range(16):
            for t in range(d_model // 16):
                zrow[r, pl.ds(t * 16, 16)] = jnp.zeros((16,), jnp.float32)
        base = pl.multiple_of(sid * RPW, 8)

        def zb(t, carry):
            pltpu.sync_copy(zrow, acc.at[pl.ds(base + t * 16, 16)])
            return carry

        lax.fori_loop(0, RPW // 16, zb, 0)
        plsc.subcore_barrier()

        def run_chunk(c, carry):
            pltpu.sync_copy(src_hbm.at[wid * n_chunks + c], src_slab)
            pltpu.sync_copy(dst_hbm.at[wid * n_chunks + c], dst_slab)
            pltpu.async_copy(hp_hbm.at[src_slab.at[0]], rows0, sem0)

            def body(jj, inner):
                j0 = 2 * jj
                j1 = j0 + 1
                pltpu.async_copy(hp_hbm.at[src_slab.at[j1]], rows1, sem1)
                pltpu.make_async_copy(
                    hp_hbm.at[src_slab.at[j0]], rows0, sem0).wait()
                pltpu.sync_copy(rows0, acc.at[dst_slab.at[j0]], add=True)
                pltpu.async_copy(hp_hbm.at[src_slab.at[j0 + 2]], rows0, sem0)
                pltpu.make_async_copy(
                    hp_hbm.at[src_slab.at[j1]], rows1, sem1).wait()
                pltpu.sync_copy(rows1, acc.at[dst_slab.at[j1]], add=True)
                return inner

            lax.fori_loop(0, (chunk - 1) // 2, body, 0)
            last = chunk - 1
            pltpu.make_async_copy(hp_hbm.at[src_slab.at[last]], rows0, sem0).wait()
            pltpu.sync_copy(rows0, acc.at[dst_slab.at[last]], add=True)
            return carry

        lax.fori_loop(0, n_chunks, run_chunk, 0)
        plsc.subcore_barrier()
        pltpu.sync_copy(acc.at[pl.ds(base, RPW)], outp.at[cid, pl.ds(base, RPW)])

    return msg_k(hp, src3, dst3)


def _lin_tc(x, w):
    n, d_in = x.shape
    d_out = w.shape[0]

    def body(x_ref, w_ref, o_ref):
        o_ref[...] = lax.dot_general(
            x_ref[...], w_ref[...], (((1,), (1,)), ((), ())),
            preferred_element_type=jnp.float32)

    return pl.pallas_call(
        body, out_shape=jax.ShapeDtypeStruct((n, d_out), jnp.float32)
    )(x, w)


def _scale_tc(h, dp):
    n, d_out = h.shape

    def body(h_ref, dp_ref, o_ref):
        deg = jnp.sum(dp_ref[...], axis=1, keepdims=True) + 1.0
        o_ref[...] = h_ref[...] * lax.rsqrt(deg)

    return pl.pallas_call(
        body, out_shape=jax.ShapeDtypeStruct((n, d_out), jnp.float32)
    )(h, dp)


def _final_tc(p0, p1, hp, dp, bias2):
    n, d_model = hp.shape

    def body(p0_ref, p1_ref, hp_ref, dp_ref, b_ref, o_ref):
        deg = jnp.sum(dp_ref[...], axis=1, keepdims=True) + 1.0
        dis = lax.rsqrt(deg)
        o_ref[...] = dis * (p0_ref[...] + p1_ref[...] + hp_ref[...]) + b_ref[...]

    return pl.pallas_call(
        body, out_shape=jax.ShapeDtypeStruct((n, d_model), jnp.float32)
    )(p0, p1, hp, dp, bias2)


def kernel(x, edge_index, W, bias):
    n, d_in = x.shape
    e = edge_index.shape[1]
    assert e % (NW * K) == 0, (e, NW, K)
    j_batches = e // (NW * K)
    src3 = edge_index[0].reshape(NW, j_batches, K)
    dst3 = edge_index[1].reshape(NW, j_batches, K)

    h = _lin_tc(x, W)
    degp = _deg_sc(dst3, n)
    dp = degp[:, :n].T
    hp = _scale_tc(h, dp)
    km, chunk = 100, 25
    assert e % (NW * km * chunk) == 0
    src4 = edge_index[0].reshape(NW * (e // (NW * km * chunk)), chunk, km)
    dst4 = edge_index[1].reshape(NW * (e // (NW * km * chunk)), chunk, km)
    parts = _msg_sc(hp, src4, dst4)
    return _final_tc(parts[0, :n], parts[1, :n], hp, dp,
                     bias.reshape(1, -1).astype(jnp.float32))
